# baseline (speedup 1.0000x reference)
import jax
import jax.numpy as jnp
from jax.experimental import pallas as pl

_GROUPS = 512
_K = 32


def _fps(xyz, npoint):
    B, N, C = xyz.shape
    batch_indices = jnp.arange(B)

    def body(i, carry):
        centroids, distance, farthest = carry
        centroids = centroids.at[:, i].set(farthest)
        centroid = xyz[batch_indices, farthest][:, None, :]
        dist = jnp.sum((xyz - centroid) ** 2, -1)
        distance = jnp.minimum(distance, dist)
        farthest = jnp.argmax(distance, -1).astype(jnp.int32)
        return (centroids, distance, farthest)

    centroids = jnp.zeros((B, npoint), dtype=jnp.int32)
    distance = jnp.full((B, N), 1e10, dtype=xyz.dtype)
    farthest = jnp.zeros((B,), dtype=jnp.int32)
    centroids, _, _ = jax.lax.fori_loop(0, npoint, body, (centroids, distance, farthest))
    return centroids


def _index_points(points, idx):
    B = points.shape[0]
    if idx.ndim == 2:
        return points[jnp.arange(B)[:, None], idx]
    else:
        return points[jnp.arange(B)[:, None, None], idx]


def _sqdist(src, dst):
    dist = -2.0 * jnp.matmul(src, jnp.transpose(dst, (0, 2, 1)))
    dist = dist + jnp.sum(src ** 2, -1)[:, :, None]
    dist = dist + jnp.sum(dst ** 2, -1)[:, None, :]
    return dist


def _passthrough(x_ref, o_ref):
    o_ref[...] = x_ref[...]


def kernel(xyz, points):
    fps_idx = _fps(xyz, _GROUPS)
    new_xyz = _index_points(xyz, fps_idx)
    new_points = _index_points(points, fps_idx)
    sqrdists = _sqdist(new_xyz, xyz)
    _, idx = jax.lax.top_k(-sqrdists, _K)
    grouped_xyz = _index_points(xyz, idx)
    grouped_points = _index_points(points, idx)
    new_xyz = pl.pallas_call(
        _passthrough,
        out_shape=jax.ShapeDtypeStruct(new_xyz.shape, new_xyz.dtype),
    )(new_xyz)
    return (new_xyz, new_points, grouped_xyz, grouped_points)


# speedup vs baseline: 3.8622x; 3.8622x over previous
import functools

import jax
import jax.numpy as jnp
from jax import lax
from jax.experimental import pallas as pl
from jax.experimental.pallas import tpu as pltpu
from jax.experimental.pallas import tpu_sc as plsc

_B = 8
_N = 16384
_G = 512
_K = 32
_QB = 8
_NBLK = _B * _G // _QB
_BIG_I = 2 ** 30


def _fps_body(x_ref, y_ref, z_ref, qx_ref, qy_ref, qz_ref, fg_ref, dist_ref):
    x = x_ref[...]
    y = y_ref[...]
    z = z_ref[...]
    iota = lax.broadcasted_iota(jnp.int32, (_B, _N), 1)
    boff = lax.broadcasted_iota(jnp.int32, (_B, 1), 0) * _N
    giota = lax.broadcasted_iota(jnp.int32, (_B, _G), 1)
    dist_ref[...] = jnp.full((_B, _N), 1e10, jnp.float32)

    def body(i, carry):
        far, qxa, qya, qza, fga = carry
        sel = iota == far
        cx = jnp.sum(jnp.where(sel, x, 0.0), axis=1, keepdims=True)
        cy = jnp.sum(jnp.where(sel, y, 0.0), axis=1, keepdims=True)
        cz = jnp.sum(jnp.where(sel, z, 0.0), axis=1, keepdims=True)
        slot = giota == i
        qxa = jnp.where(slot, cx, qxa)
        qya = jnp.where(slot, cy, qya)
        qza = jnp.where(slot, cz, qza)
        fga = jnp.where(slot, far + boff, fga)
        dx = x - cx
        dy = y - cy
        dz = z - cz
        d = (dx * dx + dy * dy) + dz * dz
        dmin = jnp.minimum(dist_ref[...], d)
        dist_ref[...] = dmin
        m = jnp.max(dmin, axis=1, keepdims=True)
        cand = jnp.where(dmin == m, iota, _BIG_I)
        far = jnp.min(cand, axis=1, keepdims=True)
        return (far, qxa, qya, qza, fga)

    zf = jnp.zeros((_B, _G), jnp.float32)
    _, qxa, qya, qza, fga = lax.fori_loop(
        0, _G, body,
        (jnp.zeros((_B, 1), jnp.int32), zf, zf, zf,
         jnp.zeros((_B, _G), jnp.int32)))
    qx_ref[...] = qxa
    qy_ref[...] = qya
    qz_ref[...] = qza
    fg_ref[...] = fga


def _knn_body(x_ref, y_ref, z_ref, p8_ref, qx_ref, qy_ref, qz_ref,
              out_ref, d_ref):
    p = pl.program_id(0)
    b = p // (_G // _QB)
    x = x_ref[0]
    y = y_ref[0]
    z = z_ref[0]
    qx = qx_ref[0]
    qy = qy_ref[0]
    qz = qz_ref[0]
    pn = (x * x + y * y) + z * z
    qn = (qx * qx + qy * qy) + qz * qz
    qpad = jnp.concatenate(
        [qx, qy, qz, jnp.zeros((_QB, 5), jnp.float32)], axis=1)
    mm = lax.dot_general(qpad, p8_ref[0], (((1,), (0,)), ((), ())),
                         precision=lax.Precision.DEFAULT)
    d_ref[...] = (-2.0 * mm + qn) + pn
    giota = b * _N + lax.broadcasted_iota(jnp.int32, (_QB, _N), 1)
    kiota = lax.broadcasted_iota(jnp.int32, (_QB, _K), 1)

    def body(j, acc):
        d = d_ref[...]
        m = jnp.min(d, axis=1, keepdims=True)
        c = jnp.where(d == m, giota, _BIG_I)
        fi = jnp.min(c, axis=1, keepdims=True)
        acc = jnp.where(kiota == j, fi, acc)
        d_ref[...] = jnp.where(c == fi, jnp.inf, d)
        return acc

    acc = lax.fori_loop(0, _K, body, jnp.zeros((_QB, _K), jnp.int32))
    out_ref[0] = acc


def _run_fps(x, y, z):
    return pl.pallas_call(
        _fps_body,
        out_shape=[
            jax.ShapeDtypeStruct((_B, _G), jnp.float32),
            jax.ShapeDtypeStruct((_B, _G), jnp.float32),
            jax.ShapeDtypeStruct((_B, _G), jnp.float32),
            jax.ShapeDtypeStruct((_B, _G), jnp.int32),
        ],
        scratch_shapes=[pltpu.VMEM((_B, _N), jnp.float32)],
    )(x, y, z)


def _run_knn(x3, y3, z3, p8, qxr, qyr, qzr):
    qspec = pl.BlockSpec((1, _QB, 1), lambda p: (p, 0, 0))
    xspec = pl.BlockSpec((1, 1, _N), lambda p: (p // (_G // _QB), 0, 0))
    pspec = pl.BlockSpec((1, 8, _N), lambda p: (p // (_G // _QB), 0, 0))
    return pl.pallas_call(
        _knn_body,
        grid=(_NBLK,),
        in_specs=[xspec, xspec, xspec, pspec, qspec, qspec, qspec],
        out_specs=pl.BlockSpec((1, _QB, _K), lambda p: (p, 0, 0)),
        out_shape=jax.ShapeDtypeStruct((_NBLK, _QB, _K), jnp.int32),
        scratch_shapes=[pltpu.VMEM((_QB, _N), jnp.float32)],
    )(x3, y3, z3, p8, qxr, qyr, qzr)


_NW = 32
_CHUNK = 128


def _wid():
    return lax.axis_index("s") * 2 + lax.axis_index("c")


@functools.cache
def _sc_mesh():
    return plsc.VectorSubcoreMesh(core_axis_name="c", subcore_axis_name="s")


@functools.cache
def _gather_rows(n_rows):
    per_w = n_rows // _NW

    @functools.partial(
        pl.kernel,
        out_type=jax.ShapeDtypeStruct((n_rows, 128), jnp.float32),
        mesh=_sc_mesh(),
        scratch_types=[
            pltpu.VMEM((_CHUNK,), jnp.int32),
            pltpu.VMEM((_CHUNK, 128), jnp.float32),
            pltpu.SemaphoreType.DMA,
        ],
    )
    def run(tab_hbm, idx_hbm, out_hbm, idx_v, rows_v, sem):
        base = _wid() * per_w

        def chunk(i, _):
            off = base + i * _CHUNK
            pltpu.sync_copy(idx_hbm.at[pl.ds(off, _CHUNK)], idx_v)
            pltpu.async_copy(tab_hbm.at[idx_v], rows_v, sem).wait()
            pltpu.sync_copy(rows_v, out_hbm.at[pl.ds(off, _CHUNK)])
            return 0

        lax.fori_loop(0, per_w // _CHUNK, chunk, 0)

    return run


def kernel(xyz, points):
    x = xyz[:, :, 0]
    y = xyz[:, :, 1]
    z = xyz[:, :, 2]
    qx, qy, qz, fpsg = _run_fps(x, y, z)

    qxr = qx.reshape(_NBLK, _QB, 1)
    qyr = qy.reshape(_NBLK, _QB, 1)
    qzr = qz.reshape(_NBLK, _QB, 1)
    x3 = x.reshape(_B, 1, _N)
    y3 = y.reshape(_B, 1, _N)
    z3 = z.reshape(_B, 1, _N)
    p8 = jnp.concatenate([xyz.transpose(0, 2, 1),
                          jnp.zeros((_B, 5, _N), jnp.float32)], axis=1)
    knng = _run_knn(x3, y3, z3, p8, qxr, qyr, qzr)

    table = jnp.pad(jnp.concatenate([points, xyz], axis=-1),
                    ((0, 0), (0, 0), (0, 61))).reshape(_B * _N, 128)

    new_rows = _gather_rows(_B * _G)(table, fpsg.reshape(-1))
    grouped_rows = _gather_rows(_B * _G * _K)(table, knng.reshape(-1))

    new_xyz = jnp.stack([qx, qy, qz], axis=-1)
    new_points = new_rows[:, :64].reshape(_B, _G, 64)
    grouped_xyz = grouped_rows[:, 64:67].reshape(_B, _G, _K, 3)
    grouped_points = grouped_rows[:, :64].reshape(_B, _G, _K, 64)
    return (new_xyz, new_points, grouped_xyz, grouped_points)
